# Initial kernel scaffold; baseline (speedup 1.0000x reference)
import jax
import jax.numpy as jnp
from jax.experimental import pallas as pl


def kernel(spatial_tokens, token_embed_weight, pos_embed):
    raise NotImplementedError("write your pallas kernel here")


# speedup vs baseline: 2.6147x; 2.6147x over previous
import functools

import jax
import jax.numpy as jnp
from jax import lax
from jax.experimental import pallas as pl
from jax.experimental.pallas import tpu as pltpu
from jax.experimental.pallas import tpu_sc as plsc

BATCH = 16
SEQ = 50
G2 = 256
D = 64
B = BATCH * SEQ * G2
NW = 32
BPW = B // NW
CH = 256
NCH = BPW // CH
LANES = 16


@functools.partial(
    pl.kernel,
    out_type=jax.ShapeDtypeStruct((B, D), jnp.float32),
    mesh=plsc.VectorSubcoreMesh(core_axis_name="c", subcore_axis_name="s"),
    scratch_types=[
        pltpu.VMEM((2, 128), jnp.int32),
        pltpu.VMEM((CH, D), jnp.float32),
        pltpu.VMEM((G2, D), jnp.float32),
        pltpu.SemaphoreType.DMA,
    ],
    compiler_params=pltpu.CompilerParams(use_tc_tiling_on_sc=False),
)
def _embed_sc(idx_hbm, tab_hbm, pos_hbm, out_hbm, idx_v, rows_v, pos_v, sem):
    wid = lax.axis_index("s") * 2 + lax.axis_index("c")
    base = pl.multiple_of(wid * BPW, CH)
    pltpu.sync_copy(pos_hbm, pos_v)

    def chunk_body(c, _):
        off = pl.multiple_of(base + c * CH, CH)
        for h in range(2):
            pltpu.sync_copy(idx_hbm.at[pl.ds(off + h * 128, 128)], idx_v.at[h])
        cps = [
            pltpu.async_copy(tab_hbm.at[idx_v.at[h]],
                             rows_v.at[pl.ds(h * 128, 128)], sem)
            for h in range(2)
        ]
        for cp in cps:
            cp.wait()

        def add_body(g, _):
            for d in range(D // LANES):
                sl = pl.ds(d * LANES, LANES)
                rows_v[g, sl] = rows_v[g, sl] + pos_v[g, sl]
            return 0

        lax.fori_loop(0, G2, add_body, 0)
        pltpu.sync_copy(rows_v, out_hbm.at[pl.ds(off, CH)])
        return 0

    lax.fori_loop(0, NCH, chunk_body, 0)


def kernel(spatial_tokens, token_embed_weight, pos_embed):
    idx = spatial_tokens.reshape(B).astype(jnp.int32)
    pos = pos_embed.reshape(G2, D)
    out = _embed_sc(idx, token_embed_weight, pos)
    return out.reshape(BATCH, SEQ, G2, D)


# speedup vs baseline: 2.8652x; 1.0958x over previous
import functools

import jax
import jax.numpy as jnp
from jax import lax
from jax.experimental import pallas as pl
from jax.experimental.pallas import tpu as pltpu
from jax.experimental.pallas import tpu_sc as plsc

BATCH = 16
SEQ = 50
G2 = 256
D = 64
B = BATCH * SEQ * G2
NW = 32
BPW = B // NW
CH = 128
NCH = BPW // CH
NBUF = 4
LANES = 16


@functools.partial(
    pl.kernel,
    out_type=jax.ShapeDtypeStruct((B, D), jnp.float32),
    mesh=plsc.VectorSubcoreMesh(core_axis_name="c", subcore_axis_name="s"),
    scratch_types=[
        pltpu.VMEM((NBUF, CH), jnp.int32),
        pltpu.VMEM((NBUF, CH, D), jnp.float32),
        pltpu.VMEM((G2, D), jnp.float32),
    ] + [pltpu.SemaphoreType.DMA] * (2 * NBUF),
    compiler_params=pltpu.CompilerParams(use_tc_tiling_on_sc=False),
)
def _embed_sc(idx_hbm, tab_hbm, pos_hbm, out_hbm, idx_v, rows_v, pos_v, *sems):
    sg = sems[:NBUF]
    ss = sems[NBUF:]
    wid = lax.axis_index("s") * 2 + lax.axis_index("c")
    base = pl.multiple_of(wid * BPW, CH)
    pltpu.sync_copy(pos_hbm, pos_v)

    def fire_gather(c, b):
        off = pl.multiple_of(base + c * CH, CH)
        pltpu.sync_copy(idx_hbm.at[pl.ds(off, CH)], idx_v.at[b])
        pltpu.async_copy(tab_hbm.at[idx_v.at[b]], rows_v.at[b], sg[b])

    def wait_gather(b):
        pltpu.make_async_copy(tab_hbm.at[idx_v.at[b]], rows_v.at[b],
                              sg[b]).wait()

    def fire_scatter(c, b):
        off = pl.multiple_of(base + c * CH, CH)
        pltpu.async_copy(rows_v.at[b], out_hbm.at[pl.ds(off, CH)], ss[b])

    def wait_scatter(b):
        pltpu.make_async_copy(rows_v.at[b], out_hbm.at[pl.ds(base, CH)],
                              ss[b]).wait()

    def do_add(b, parity):
        pb = parity * CH

        @plsc.parallel_loop(0, CH, step=1, unroll=8)
        def _(g):
            for d in range(D // LANES):
                sl = pl.ds(d * LANES, LANES)
                rows_v[b, g, sl] = rows_v[b, g, sl] + pos_v[pb + g, sl]

    for c in range(NBUF - 1):
        fire_gather(c, c)

    def ring_body(i, _):
        for b in range(NBUF):
            c = NBUF * i + b
            b3 = (b + NBUF - 1) % NBUF

            @pl.when(c + NBUF - 1 < NCH)
            def _():
                @pl.when(c >= 1)
                def _():
                    wait_scatter(b3)

                fire_gather(c + NBUF - 1, b3)

            wait_gather(b)
            do_add(b, b & 1)
            fire_scatter(c, b)
        return 0

    full = NCH // NBUF
    lax.fori_loop(0, full, ring_body, 0)
    for c in range(full * NBUF, NCH):
        b = c % NBUF
        wait_gather(b)
        do_add(b, c & 1)
        fire_scatter(c, b)
    for b in range(NBUF):
        wait_scatter(b)


def kernel(spatial_tokens, token_embed_weight, pos_embed):
    idx = spatial_tokens.reshape(B).astype(jnp.int32)
    pos = pos_embed.reshape(G2, D)
    out = _embed_sc(idx, token_embed_weight, pos)
    return out.reshape(BATCH, SEQ, G2, D)


# speedup vs baseline: 3.2546x; 1.1359x over previous
import functools

import jax
import jax.numpy as jnp
from jax import lax
from jax.experimental import pallas as pl
from jax.experimental.pallas import tpu as pltpu
from jax.experimental.pallas import tpu_sc as plsc

BATCH = 16
SEQ = 50
G2 = 256
D = 64
DP = 128
CH = 128
NW = 32
NCHW = SEQ
NBUF = 2
LANES = 16


@functools.partial(
    pl.kernel,
    out_type=jax.ShapeDtypeStruct((BATCH, SEQ, G2, D), jnp.float32),
    mesh=plsc.VectorSubcoreMesh(core_axis_name="c", subcore_axis_name="s"),
    scratch_types=[
        pltpu.VMEM((NBUF, CH), jnp.int32),
        pltpu.VMEM((NBUF, CH, DP), jnp.float32),
        pltpu.VMEM((NBUF, CH, D), jnp.float32),
        pltpu.VMEM((G2, D), jnp.float32),
    ] + [pltpu.SemaphoreType.DMA] * (2 * NBUF),
)
def _embed_sc(idx_hbm, tab_hbm, pos_hbm, out_hbm,
              idx_v, rows_v, outb_v, pos_v, *sems):
    sg = sems[:NBUF]
    ss = sems[NBUF:]
    wid = lax.axis_index("s") * 2 + lax.axis_index("c")
    bb = wid // 2
    ss0 = (wid % 2) * (SEQ // 2)
    pltpu.sync_copy(pos_hbm.at[0], pos_v)

    def fire_gather(i, par, p):
        gsl = pl.ds(par * CH, CH)
        pltpu.sync_copy(idx_hbm.at[bb, ss0 + i, gsl], idx_v.at[p])
        pltpu.async_copy(tab_hbm.at[idx_v.at[p]], rows_v.at[p], sg[p])

    def wait_gather(p):
        pltpu.make_async_copy(tab_hbm.at[idx_v.at[p]], rows_v.at[p],
                              sg[p]).wait()

    def fire_store(i, par, p):
        pltpu.async_copy(outb_v.at[p],
                         out_hbm.at[bb, ss0 + i, pl.ds(par * CH, CH)], ss[p])

    def wait_store(p):
        pltpu.make_async_copy(outb_v.at[p],
                              out_hbm.at[0, 0, pl.ds(0, CH)], ss[p]).wait()

    def do_add(par, p):
        @plsc.parallel_loop(0, CH, step=1, unroll=8)
        def _(g):
            for d in range(D // LANES):
                sl = pl.ds(d * LANES, LANES)
                outb_v[p, g, sl] = rows_v[p, g, sl] + pos_v[par * CH + g, sl]

    fire_gather(0, 0, 0)

    def ring_body(i, _):
        for b in range(NBUF):
            c = NBUF * i + b
            q = 1 - b

            @pl.when(c + 1 < NCHW)
            def _():
                @pl.when(c >= 1)
                def _():
                    wait_store(q)

                fire_gather(i + b, 1 - b, q)

            wait_gather(b)
            do_add(b, b)
            fire_store(i, b, b)
        return 0

    lax.fori_loop(0, NCHW // NBUF, ring_body, 0)
    for p in range(NBUF):
        wait_store(p)


def kernel(spatial_tokens, token_embed_weight, pos_embed):
    tab128 = jnp.pad(token_embed_weight, ((0, 0), (0, DP - D)))
    return _embed_sc(spatial_tokens.astype(jnp.int32), tab128, pos_embed)
